# baseline (device time: 23555 ns/iter reference)
import jax
import jax.numpy as jnp
from jax import lax
from jax.experimental import pallas as pl
from jax.experimental.pallas import tpu as pltpu

C = 8


def kernel(partial, resid, gamma):
    m, d = resid.shape
    rc = m // C

    def body(
        p_hbm, resid_hbm, gamma_ref, out_hbm,
        pbuf, rbuf, obuf, send_q, recv_q, send_s, recv_s,
        p_sems, r_sems, o_sems, sq_sems, rq_sems, ss_sems, rs_sems,
    ):
        my_x = lax.axis_index("x")
        my_y = lax.axis_index("y")
        x_peer = (1 - my_x, my_y)

        barrier_sem = pltpu.get_barrier_semaphore()
        pl.semaphore_signal(
            barrier_sem, inc=1,
            device_id=x_peer, device_id_type=pl.DeviceIdType.MESH,
        )
        pl.semaphore_wait(barrier_sem, 1)

        p_copies = []
        r_copies = []
        for c in range(C):
            rows = pl.ds(c * rc, rc)
            cp = pltpu.make_async_copy(
                p_hbm.at[0, rows, :], pbuf.at[c], p_sems.at[c]
            )
            cp.start()
            p_copies.append(cp)
            cr = pltpu.make_async_copy(
                resid_hbm.at[rows, :], rbuf.at[c], r_sems.at[c]
            )
            cr.start()
            r_copies.append(cr)

        data_rdmas = []
        scale_rdmas = []
        for c in range(C):
            p_copies[c].wait()
            p = pbuf[c, :, :]
            absmax = jnp.max(jnp.abs(p))
            send_s[c, :, :] = jnp.reshape(absmax * (1.0 / 127.0), (1, 1))
            inv = 127.0 / jnp.maximum(absmax, 1e-30)
            send_q[c, :, :] = jnp.rint(p * inv).astype(jnp.int8)
            r_s = pltpu.make_async_remote_copy(
                src_ref=send_s.at[c],
                dst_ref=recv_s.at[c],
                send_sem=ss_sems.at[c],
                recv_sem=rs_sems.at[c],
                device_id=x_peer,
                device_id_type=pl.DeviceIdType.MESH,
            )
            r_q = pltpu.make_async_remote_copy(
                src_ref=send_q.at[c],
                dst_ref=recv_q.at[c],
                send_sem=sq_sems.at[c],
                recv_sem=rq_sems.at[c],
                device_id=x_peer,
                device_id_type=pl.DeviceIdType.MESH,
            )
            r_s.start()
            r_q.start()
            scale_rdmas.append(r_s)
            data_rdmas.append(r_q)
            r_copies[c].wait()
            rbuf[c, :, :] = p + rbuf[c, :, :]

        o_copies = []
        for c in range(C):
            scale_rdmas[c].wait_recv()
            data_rdmas[c].wait_recv()
            theirs = recv_q[c, :, :].astype(jnp.float32) * recv_s[c, :, :]
            y = rbuf[c, :, :] + theirs
            rms = jnp.sqrt(jnp.mean(y * y, axis=-1, keepdims=True) + 1e-6)
            obuf[c, :, :] = (y / rms) * gamma_ref[...][None, :]
            co = pltpu.make_async_copy(
                obuf.at[c], out_hbm.at[pl.ds(c * rc, rc), :], o_sems.at[c]
            )
            co.start()
            o_copies.append(co)

        for c in range(C):
            o_copies[c].wait()
            scale_rdmas[c].wait_send()
            data_rdmas[c].wait_send()

    return pl.pallas_call(
        body,
        out_shape=jax.ShapeDtypeStruct((m, d), jnp.float32),
        in_specs=[
            pl.BlockSpec(memory_space=pl.MemorySpace.ANY),
            pl.BlockSpec(memory_space=pl.MemorySpace.ANY),
            pl.BlockSpec(memory_space=pltpu.VMEM),
        ],
        out_specs=pl.BlockSpec(memory_space=pl.MemorySpace.ANY),
        scratch_shapes=[
            pltpu.VMEM((C, rc, d), jnp.float32),
            pltpu.VMEM((C, rc, d), jnp.float32),
            pltpu.VMEM((C, rc, d), jnp.float32),
            pltpu.VMEM((C, rc, d), jnp.int8),
            pltpu.VMEM((C, rc, d), jnp.int8),
            pltpu.VMEM((C, 1, 1), jnp.float32),
            pltpu.VMEM((C, 1, 1), jnp.float32),
            pltpu.SemaphoreType.DMA((C,)),
            pltpu.SemaphoreType.DMA((C,)),
            pltpu.SemaphoreType.DMA((C,)),
            pltpu.SemaphoreType.DMA((C,)),
            pltpu.SemaphoreType.DMA((C,)),
            pltpu.SemaphoreType.DMA((C,)),
            pltpu.SemaphoreType.DMA((C,)),
        ],
        compiler_params=pltpu.CompilerParams(collective_id=0),
    )(partial, resid, gamma)


# device time: 22654 ns/iter; 1.0398x vs baseline; 1.0398x over previous
import jax
import jax.numpy as jnp
from jax import lax
from jax.experimental import pallas as pl
from jax.experimental.pallas import tpu as pltpu

C = 8


def kernel(partial, resid, gamma):
    m, d = resid.shape
    rc = m // C

    def body(
        p_ref, resid_ref, gamma_ref, out_ref,
        send_q, recv_q, send_s, recv_s, local_sum,
        sq_sems, rq_sems, ss_sems, rs_sems,
    ):
        my_x = lax.axis_index("x")
        my_y = lax.axis_index("y")
        x_peer = (1 - my_x, my_y)

        barrier_sem = pltpu.get_barrier_semaphore()
        pl.semaphore_signal(
            barrier_sem, inc=1,
            device_id=x_peer, device_id_type=pl.DeviceIdType.MESH,
        )
        pl.semaphore_wait(barrier_sem, 1)

        data_rdmas = []
        scale_rdmas = []
        for c in range(C):
            rows = slice(c * rc, (c + 1) * rc)
            p = p_ref[0, rows, :]
            absmax = jnp.max(jnp.abs(p))
            send_s[c, :, :] = jnp.reshape(absmax * (1.0 / 127.0), (1, 1))
            inv = 127.0 / jnp.maximum(absmax, 1e-30)
            send_q[c, :, :] = jnp.rint(p * inv).astype(jnp.int8)
            r_s = pltpu.make_async_remote_copy(
                src_ref=send_s.at[c],
                dst_ref=recv_s.at[c],
                send_sem=ss_sems.at[c],
                recv_sem=rs_sems.at[c],
                device_id=x_peer,
                device_id_type=pl.DeviceIdType.MESH,
            )
            r_q = pltpu.make_async_remote_copy(
                src_ref=send_q.at[c],
                dst_ref=recv_q.at[c],
                send_sem=sq_sems.at[c],
                recv_sem=rq_sems.at[c],
                device_id=x_peer,
                device_id_type=pl.DeviceIdType.MESH,
            )
            r_s.start()
            r_q.start()
            scale_rdmas.append(r_s)
            data_rdmas.append(r_q)
            local_sum[c, :, :] = p + resid_ref[rows, :]

        for c in range(C):
            scale_rdmas[c].wait_recv()
            data_rdmas[c].wait_recv()
            theirs = recv_q[c, :, :].astype(jnp.float32) * recv_s[c, :, :]
            y = local_sum[c, :, :] + theirs
            rms = jnp.sqrt(jnp.mean(y * y, axis=-1, keepdims=True) + 1e-6)
            out_ref[c * rc : (c + 1) * rc, :] = (y / rms) * gamma_ref[...][None, :]

        for c in range(C):
            scale_rdmas[c].wait_send()
            data_rdmas[c].wait_send()

    return pl.pallas_call(
        body,
        out_shape=jax.ShapeDtypeStruct((m, d), jnp.float32),
        in_specs=[
            pl.BlockSpec(memory_space=pltpu.VMEM),
            pl.BlockSpec(memory_space=pltpu.VMEM),
            pl.BlockSpec(memory_space=pltpu.VMEM),
        ],
        out_specs=pl.BlockSpec(memory_space=pltpu.VMEM),
        scratch_shapes=[
            pltpu.VMEM((C, rc, d), jnp.int8),
            pltpu.VMEM((C, rc, d), jnp.int8),
            pltpu.VMEM((C, 1, 1), jnp.float32),
            pltpu.VMEM((C, 1, 1), jnp.float32),
            pltpu.VMEM((C, rc, d), jnp.float32),
            pltpu.SemaphoreType.DMA((C,)),
            pltpu.SemaphoreType.DMA((C,)),
            pltpu.SemaphoreType.DMA((C,)),
            pltpu.SemaphoreType.DMA((C,)),
        ],
        compiler_params=pltpu.CompilerParams(collective_id=0),
    )(partial, resid, gamma)
